# baseline (speedup 1.0000x reference)
import functools

import jax
import jax.numpy as jnp
from jax.experimental import pallas as pl
from jax.experimental.pallas import tpu as pltpu

N = 10000
E = 160000
D = 256
H = 8
F = D // H


def _mm_kernel(x_ref, w_ref, o_ref):
    o_ref[...] = jnp.dot(x_ref[...], w_ref[...],
                         preferred_element_type=jnp.float32)


def _matmul(x, w):
    m = x.shape[0]
    blk = 1000
    return pl.pallas_call(
        _mm_kernel,
        grid=(m // blk,),
        in_specs=[
            pl.BlockSpec((blk, D), lambda i: (i, 0)),
            pl.BlockSpec((D, D), lambda i: (0, 0)),
        ],
        out_specs=pl.BlockSpec((blk, D), lambda i: (i, 0)),
        out_shape=jax.ShapeDtypeStruct((m, D), jnp.float32),
    )(x, w)


def _layernorm(x, g, b):
    mu = jnp.mean(x, axis=-1, keepdims=True)
    var = jnp.var(x, axis=-1, keepdims=True)
    return (x - mu) / jnp.sqrt(var + 1e-5) * g + b


def _gat_conv(x, src, dst, W, al, ar, b):
    feat = _matmul(x, W).reshape(N, H, F)
    el = jnp.sum(feat * al[None, :, :], axis=-1)
    er = jnp.sum(feat * ar[None, :, :], axis=-1)
    e = jax.nn.leaky_relu(el[src] + er[dst], negative_slope=0.2)
    emax = jax.ops.segment_max(e, dst, num_segments=N)
    emax = jnp.where(jnp.isfinite(emax), emax, 0.0)
    ee = jnp.exp(e - emax[dst])
    denom = jax.ops.segment_sum(ee, dst, num_segments=N)
    alpha = ee / (denom[dst] + 1e-9)
    msg = feat[src] * alpha[:, :, None]
    out = jax.ops.segment_sum(msg, dst, num_segments=N)
    out = out + b.reshape(1, H, F)
    out = jax.nn.relu(out)
    return out.reshape(N, D)


def _ff(x, g, bn, W1, b1, W2, b2):
    h = _layernorm(x, g, bn)
    h = jax.nn.relu(_matmul(h, W1) + b1)
    h = _matmul(h, W2) + b2
    return x + h


def kernel(x, edge_index, conv_W0, conv_al0, conv_ar0, conv_b0, ln_g0, ln_b0, ff_W1_0, ff_b1_0, ff_W2_0, ff_b2_0, conv_W1, conv_al1, conv_ar1, conv_b1, ln_g1, ln_b1, ff_W1_1, ff_b1_1, ff_W2_1, ff_b2_1):
    src = edge_index[0]
    dst = edge_index[1]
    h = _gat_conv(x, src, dst, conv_W0, conv_al0, conv_ar0, conv_b0)
    h = _ff(h, ln_g0, ln_b0, ff_W1_0, ff_b1_0, ff_W2_0, ff_b2_0)
    h = _gat_conv(h, src, dst, conv_W1, conv_al1, conv_ar1, conv_b1)
    h = _ff(h, ln_g1, ln_b1, ff_W1_1, ff_b1_1, ff_W2_1, ff_b2_1)
    return h


# speedup vs baseline: 28.9326x; 28.9326x over previous
import jax
import jax.numpy as jnp
from jax import lax
from jax.experimental import pallas as pl
from jax.experimental.pallas import tpu as pltpu
from jax.experimental.pallas import tpu_sc as plsc

N = 10000
E = 160000
D = 256
H = 8
F = D // H
NC = 2
NS = 16
DH = D // NC
HC = H // NC
EPT = E // NS
B = 80
NCH = N // B
ECH = EPT // B
BLK = 1000
PN = 10112
PB = 128
DQ = D // 4
HQ = 2


def _proj_kernel(x_ref, w_ref, alr_ref, feat_ref, elr_ref, bmax_ref):
    feat = jnp.dot(x_ref[...], w_ref[...], preferred_element_type=jnp.float32)
    feat_ref[...] = feat
    elr = jnp.dot(feat, alr_ref[...], preferred_element_type=jnp.float32)
    elr_ref[...] = elr
    bmax_ref[...] = jnp.max(elr, axis=0, keepdims=True)[None]


def _proj(x, w, alr):
    grid = N // BLK
    return pl.pallas_call(
        _proj_kernel,
        grid=(grid,),
        in_specs=[
            pl.BlockSpec((BLK, D), lambda i: (i, 0)),
            pl.BlockSpec((D, D), lambda i: (0, 0)),
            pl.BlockSpec((D, 2 * H), lambda i: (0, 0)),
        ],
        out_specs=[
            pl.BlockSpec((BLK, D), lambda i: (i, 0)),
            pl.BlockSpec((BLK, 2 * H), lambda i: (i, 0)),
            pl.BlockSpec((1, 1, 2 * H), lambda i: (i, 0, 0)),
        ],
        out_shape=[
            jax.ShapeDtypeStruct((N, D), jnp.float32),
            jax.ShapeDtypeStruct((N, 2 * H), jnp.float32),
            jax.ShapeDtypeStruct((grid, 1, 2 * H), jnp.float32),
        ],
    )(x, w, alr)


def _ffn_kernel(a_ref, b_ref, c_ref, d_ref, g_ref, bn_ref, w1_ref, b1_ref,
                w2_ref, b2_ref, o_ref):
    h = jnp.concatenate([a_ref[...], b_ref[...], c_ref[...], d_ref[...]],
                        axis=1)
    mu = jnp.mean(h, axis=-1, keepdims=True)
    xc = h - mu
    var = jnp.mean(xc * xc, axis=-1, keepdims=True)
    ln = xc * lax.rsqrt(var + 1e-5) * g_ref[...] + bn_ref[...]
    f1 = jnp.maximum(
        jnp.dot(ln, w1_ref[...], preferred_element_type=jnp.float32)
        + b1_ref[...], 0.0)
    f2 = jnp.dot(f1, w2_ref[...], preferred_element_type=jnp.float32) \
        + b2_ref[...]
    o_ref[...] = h + f2


def _ffn(o4, g, bn, w1, b1, w2, b2):
    grid = N // BLK
    vec = pl.BlockSpec((1, D), lambda i: (0, 0))
    mat = pl.BlockSpec((D, D), lambda i: (0, 0))
    quad = pl.BlockSpec((BLK, DQ), lambda i: (i, 0))
    return pl.pallas_call(
        _ffn_kernel,
        grid=(grid,),
        in_specs=[quad, quad, quad, quad, vec, vec, mat, vec, mat, vec],
        out_specs=pl.BlockSpec((BLK, D), lambda i: (i, 0)),
        out_shape=jax.ShapeDtypeStruct((N, D), jnp.float32),
    )(o4[0], o4[1], o4[2], o4[3], g.reshape(1, D), bn.reshape(1, D), w1,
      b1.reshape(1, D), w2, b2.reshape(1, D))


def _conv_sc_body(feat_hbm, elt_hbm, src_hbm, dst_hbm, c_hbm, b_hbm, out_hbm,
                  elv, srcv, dstv, rowbuf, pb0, pb1, gidx, didx,
                  cv, bv, num_sp, dn0, dn1, sem):
    c = lax.axis_index("c")
    s = lax.axis_index("s")
    pbs = [pb0, pb1]
    dens = [dn0, dn1]
    zero16 = jnp.zeros((16,), jnp.float32)

    pltpu.sync_copy(src_hbm.at[pl.ds(s * EPT, EPT)], srcv)
    pltpu.sync_copy(dst_hbm.at[pl.ds(s * EPT, EPT)], dstv)
    pltpu.sync_copy(c_hbm, cv)

    for k in range(2):
        q = 2 * k + c
        for r in range(HQ):
            pltpu.sync_copy(elt_hbm.at[pl.ds((2 * k * HQ) * N + c * HQ * N + r * N, N)],
                            elv.at[pl.ds(r * PN, N)])
            pltpu.sync_copy(elt_hbm.at[pl.ds((H + 2 * k * HQ) * N + c * HQ * N + r * N, N)],
                            elv.at[pl.ds((HQ + r) * PN, N)])
        pltpu.sync_copy(b_hbm.at[pl.ds((2 * k + c) * DQ, DQ)], bv)

        def _zero_row(r, carry):
            for t in range(DQ // 16):
                rowbuf[r, pl.ds(t * 16, 16)] = zero16
            return carry
        lax.fori_loop(0, B, _zero_row, 0)
        for v in range(B // 16):
            for h in range(HQ):
                pbs[h][pl.ds(v * 16, 16)] = zero16

        def _zero_chunk(j, carry):
            idx = j * NS + s

            @pl.when(idx < NCH)
            def _():
                nb = idx * B
                pltpu.sync_copy(rowbuf, num_sp.at[pl.ds(nb, B)])
                for h in range(HQ):
                    pltpu.sync_copy(pbs[h].at[pl.ds(0, B)],
                                    dens[h].at[pl.ds(nb, B)])
            return carry
        lax.fori_loop(0, (NCH + NS - 1) // NS, _zero_chunk, 0)
        plsc.subcore_barrier()

        def _edge_chunk(j, carry):
            eb = j * B
            for v in range(B // 16):
                sv = srcv[pl.ds(eb + v * 16, 16)]
                dv = dstv[pl.ds(eb + v * 16, 16)]
                gidx[pl.ds(v * 16, 16)] = sv * 4 + q
                didx[pl.ds(v * 16, 16)] = dv
            gat = pltpu.async_copy(feat_hbm.at[gidx], rowbuf, sem)
            for h in range(HQ):
                ch = plsc.load_gather(
                    cv, [jnp.full((16,), 2 * q + h, jnp.int32)])
                for v in range(B // 16):
                    sv = srcv[pl.ds(eb + v * 16, 16)]
                    dv = dstv[pl.ds(eb + v * 16, 16)]
                    el = plsc.load_gather(elv, [sv + h * PN])
                    er = plsc.load_gather(elv, [dv + (HQ + h) * PN])
                    e = el + er
                    e = jnp.where(e >= 0.0, e, 0.2 * e)
                    p = jnp.exp(e - ch)
                    pbs[h][pl.ds(v * 16, 16)] = p
            for h in range(HQ):
                pltpu.sync_copy(pbs[h].at[pl.ds(0, B)], dens[h].at[didx],
                                add=True)
            gat.wait()

            def _scale(i, carry2):
                for h in range(HQ):
                    pv = plsc.load_gather(
                        pbs[h], [jnp.full((16,), i, jnp.int32)])
                    for t in range(2):
                        sl = pl.ds(h * 2 * 16 + t * 16, 16)
                        rowbuf[i, sl] = rowbuf[i, sl] * pv
                return carry2
            lax.fori_loop(0, B, _scale, 0)
            pltpu.sync_copy(rowbuf, num_sp.at[didx], add=True)
            return carry
        lax.fori_loop(0, ECH, _edge_chunk, 0)
        plsc.subcore_barrier()

        def _fin_chunk(j, carry):
            idx = j * NS + s

            @pl.when(idx < NCH)
            def _():
                nb = idx * B
                pltpu.sync_copy(num_sp.at[pl.ds(nb, B)], rowbuf)
                for h in range(HQ):
                    pltpu.sync_copy(dens[h].at[pl.ds(nb, B)],
                                    pbs[h].at[pl.ds(0, B)])

                def _node(i, carry2):
                    for h in range(HQ):
                        dv = plsc.load_gather(
                            pbs[h], [jnp.full((16,), i, jnp.int32)]) + 1e-9
                        for t in range(2):
                            sl = pl.ds(h * 2 * 16 + t * 16, 16)
                            qv = rowbuf[i, sl] / dv + bv[sl]
                            rowbuf[i, sl] = jnp.maximum(qv, 0.0)
                    return carry2
                lax.fori_loop(0, B, _node, 0)
                pltpu.sync_copy(rowbuf, out_hbm.at[q, pl.ds(nb, B)])
            return carry
        lax.fori_loop(0, (NCH + NS - 1) // NS, _fin_chunk, 0)
        plsc.subcore_barrier()


def _conv_sc(feat4, elt, src, dst, cvec, b):
    mesh = plsc.VectorSubcoreMesh(core_axis_name="c", subcore_axis_name="s")
    return pl.kernel(
        _conv_sc_body,
        out_type=jax.ShapeDtypeStruct((4, N, DQ), jnp.float32),
        mesh=mesh,
        compiler_params=pltpu.CompilerParams(needs_layout_passes=False, use_tc_tiling_on_sc=False),
        scratch_types=[
            pltpu.VMEM((2 * HQ * PN,), jnp.float32),
            pltpu.VMEM((EPT,), jnp.int32),
            pltpu.VMEM((EPT,), jnp.int32),
            pltpu.VMEM((B, DQ), jnp.float32),
            pltpu.VMEM((PB,), jnp.float32),
            pltpu.VMEM((PB,), jnp.float32),
            pltpu.VMEM((B,), jnp.int32),
            pltpu.VMEM((B,), jnp.int32),
            pltpu.VMEM((128,), jnp.float32),
            pltpu.VMEM((DQ,), jnp.float32),
            pltpu.VMEM_SHARED((N, DQ), jnp.float32),
            pltpu.VMEM_SHARED((PN,), jnp.float32),
            pltpu.VMEM_SHARED((PN,), jnp.float32),
            pltpu.SemaphoreType.DMA,
        ],
    )(feat4, elt, src, dst, cvec, b)


def _alr_mat(al, ar):
    d = jnp.arange(D)
    hh = d // F
    ff = d % F
    ml = jnp.zeros((D, H), jnp.float32).at[d, hh].set(al[hh, ff])
    mr = jnp.zeros((D, H), jnp.float32).at[d, hh].set(ar[hh, ff])
    return jnp.concatenate([ml, mr], axis=1)


def _gat_layer(h, src, dst, W, al, ar, b):
    feat, elr, bmax = _proj(h, W, _alr_mat(al, ar))
    m = jnp.max(bmax, axis=(0, 1))
    c8 = jnp.maximum(m[:H] + m[H:], 0.0)
    cvec = jnp.concatenate([c8, jnp.zeros((128 - H,), jnp.float32)])
    return _conv_sc(feat.reshape(4 * N, DQ), elr.T.reshape(2 * H * N),
                    src, dst, cvec, b)


def kernel(x, edge_index, conv_W0, conv_al0, conv_ar0, conv_b0, ln_g0, ln_b0, ff_W1_0, ff_b1_0, ff_W2_0, ff_b2_0, conv_W1, conv_al1, conv_ar1, conv_b1, ln_g1, ln_b1, ff_W1_1, ff_b1_1, ff_W2_1, ff_b2_1):
    src = edge_index[0]
    dst = edge_index[1]
    o = _gat_layer(x, src, dst, conv_W0, conv_al0, conv_ar0, conv_b0)
    h = _ffn(o, ln_g0, ln_b0, ff_W1_0, ff_b1_0, ff_W2_0, ff_b2_0)
    o = _gat_layer(h, src, dst, conv_W1, conv_al1, conv_ar1, conv_b1)
    h = _ffn(o, ln_g1, ln_b1, ff_W1_1, ff_b1_1, ff_W2_1, ff_b2_1)
    return h


# speedup vs baseline: 52.1216x; 1.8015x over previous
import jax
import jax.numpy as jnp
from jax import lax
from jax.experimental import pallas as pl
from jax.experimental.pallas import tpu as pltpu
from jax.experimental.pallas import tpu_sc as plsc

N = 10000
E = 160000
D = 256
H = 8
F = D // H
NC = 2
NS = 16
DH = D // NC
HC = H // NC
EPT = E // NS
B = 80
NCH = N // B
ECH = EPT // B
BLK = 1000
PN = 10112
PB = 128
DQ = D // 4
HQ = 2


def _proj_kernel(x_ref, w_ref, alr_ref, feat_ref, elr_ref, bmax_ref):
    feat = jnp.dot(x_ref[...], w_ref[...], preferred_element_type=jnp.float32)
    feat_ref[...] = feat
    elr = jnp.dot(feat, alr_ref[...], preferred_element_type=jnp.float32)
    elr_ref[...] = elr
    bmax_ref[...] = jnp.max(elr, axis=0, keepdims=True)[None]


def _proj(x, w, alr):
    grid = N // BLK
    return pl.pallas_call(
        _proj_kernel,
        grid=(grid,),
        in_specs=[
            pl.BlockSpec((BLK, D), lambda i: (i, 0)),
            pl.BlockSpec((D, D), lambda i: (0, 0)),
            pl.BlockSpec((D, 2 * H), lambda i: (0, 0)),
        ],
        out_specs=[
            pl.BlockSpec((BLK, D), lambda i: (i, 0)),
            pl.BlockSpec((BLK, 2 * H), lambda i: (i, 0)),
            pl.BlockSpec((1, 1, 2 * H), lambda i: (i, 0, 0)),
        ],
        out_shape=[
            jax.ShapeDtypeStruct((N, D), jnp.float32),
            jax.ShapeDtypeStruct((N, 2 * H), jnp.float32),
            jax.ShapeDtypeStruct((grid, 1, 2 * H), jnp.float32),
        ],
    )(x, w, alr)


def _ffn_kernel(a_ref, b_ref, c_ref, d_ref, g_ref, bn_ref, w1_ref, b1_ref,
                w2_ref, b2_ref, o_ref):
    h = jnp.concatenate([a_ref[...], b_ref[...], c_ref[...], d_ref[...]],
                        axis=1)
    mu = jnp.mean(h, axis=-1, keepdims=True)
    xc = h - mu
    var = jnp.mean(xc * xc, axis=-1, keepdims=True)
    ln = xc * lax.rsqrt(var + 1e-5) * g_ref[...] + bn_ref[...]
    f1 = jnp.maximum(
        jnp.dot(ln, w1_ref[...], preferred_element_type=jnp.float32)
        + b1_ref[...], 0.0)
    f2 = jnp.dot(f1, w2_ref[...], preferred_element_type=jnp.float32) \
        + b2_ref[...]
    o_ref[...] = h + f2


def _ffn(o4, g, bn, w1, b1, w2, b2):
    grid = N // BLK
    vec = pl.BlockSpec((1, D), lambda i: (0, 0))
    mat = pl.BlockSpec((D, D), lambda i: (0, 0))
    quad = pl.BlockSpec((BLK, DQ), lambda i: (i, 0))
    return pl.pallas_call(
        _ffn_kernel,
        grid=(grid,),
        in_specs=[quad, quad, quad, quad, vec, vec, mat, vec, mat, vec],
        out_specs=pl.BlockSpec((BLK, D), lambda i: (i, 0)),
        out_shape=jax.ShapeDtypeStruct((N, D), jnp.float32),
    )(o4[0], o4[1], o4[2], o4[3], g.reshape(1, D), bn.reshape(1, D), w1,
      b1.reshape(1, D), w2, b2.reshape(1, D))


def _conv_sc_body(feat_hbm, elt_hbm, src_hbm, dst_hbm, c_hbm, b_hbm, out_hbm,
                  elv, srcv, dstv, rb0, rb1, pb0, pb1, gi0, gi1, di0, di1,
                  cv, bv, num_sp, dn0, dn1, sem0, sem1):
    c = lax.axis_index("c")
    s = lax.axis_index("s")
    pbs = [pb0, pb1]
    dens = [dn0, dn1]
    bufs = [(rb0, gi0, di0, sem0), (rb1, gi1, di1, sem1)]
    zero16 = jnp.zeros((16,), jnp.float32)

    pltpu.sync_copy(src_hbm.at[pl.ds(s * EPT, EPT)], srcv)
    pltpu.sync_copy(dst_hbm.at[pl.ds(s * EPT, EPT)], dstv)
    pltpu.sync_copy(c_hbm, cv)

    for k in range(2):
        q = 2 * k + c
        for r in range(HQ):
            pltpu.sync_copy(
                elt_hbm.at[pl.ds((2 * k * HQ) * N + c * HQ * N + r * N, N)],
                elv.at[pl.ds(r * PN, N)])
            pltpu.sync_copy(
                elt_hbm.at[pl.ds((H + 2 * k * HQ) * N + c * HQ * N + r * N, N)],
                elv.at[pl.ds((HQ + r) * PN, N)])
        pltpu.sync_copy(b_hbm.at[pl.ds((2 * k + c) * DQ, DQ)], bv)

        def _zero_row(r, carry):
            for t in range(DQ // 16):
                rb0[r, pl.ds(t * 16, 16)] = zero16
            return carry
        lax.fori_loop(0, B, _zero_row, 0)
        for v in range(B // 16):
            for h in range(HQ):
                pbs[h][pl.ds(v * 16, 16)] = zero16

        def _zero_chunk(j, carry):
            idx = j * NS + s

            @pl.when(idx < NCH)
            def _():
                nb = idx * B
                pltpu.sync_copy(rb0, num_sp.at[pl.ds(nb, B)])
                for h in range(HQ):
                    pltpu.sync_copy(pbs[h].at[pl.ds(0, B)],
                                    dens[h].at[pl.ds(nb, B)])
            return carry
        lax.fori_loop(0, (NCH + NS - 1) // NS, _zero_chunk, 0)
        plsc.subcore_barrier()

        def _build_issue(j, par):
            rb, gi, di, sm = bufs[par]
            for v in range(B // 16):
                sv = srcv[pl.ds(j * B + v * 16, 16)]
                dv = dstv[pl.ds(j * B + v * 16, 16)]
                gi[pl.ds(v * 16, 16)] = sv * 4 + q
                di[pl.ds(v * 16, 16)] = dv
            pltpu.async_copy(feat_hbm.at[gi], rb, sm)

        def _consume(j, par):
            rb, gi, di, sm = bufs[par]
            for h in range(HQ):
                ch = plsc.load_gather(
                    cv, [jnp.full((16,), 2 * q + h, jnp.int32)])
                for v in range(B // 16):
                    sv = srcv[pl.ds(j * B + v * 16, 16)]
                    dv = dstv[pl.ds(j * B + v * 16, 16)]
                    el = plsc.load_gather(elv, [sv + h * PN])
                    er = plsc.load_gather(elv, [dv + (HQ + h) * PN])
                    e = el + er
                    e = jnp.where(e >= 0.0, e, 0.2 * e)
                    p = jnp.exp(e - ch)
                    pbs[h][pl.ds(v * 16, 16)] = p
            for h in range(HQ):
                pltpu.sync_copy(pbs[h].at[pl.ds(0, B)], dens[h].at[di],
                                add=True)
            pltpu.make_async_copy(feat_hbm.at[gi], rb, sm).wait()

            def _scale_grp(g, carry2):
                p16 = [pbs[h][pl.ds(g * 16, 16)] for h in range(HQ)]
                base = g * 16
                for ee in range(16):
                    for h in range(HQ):
                        pv = jnp.full((16,), p16[h][ee])
                        for t in range(2):
                            sl = pl.ds(h * 32 + t * 16, 16)
                            rb[base + ee, sl] = rb[base + ee, sl] * pv
                return carry2
            lax.fori_loop(0, B // 16, _scale_grp, 0)
            pltpu.sync_copy(rb, num_sp.at[di], add=True)

        _build_issue(0, 0)

        def _pair(jj, carry):
            j0 = 2 * jj
            _build_issue(j0 + 1, 1)
            _consume(j0, 0)
            _build_issue(j0 + 2, 0)
            _consume(j0 + 1, 1)
            return carry
        lax.fori_loop(0, (ECH - 1) // 2, _pair, 0)
        _consume(ECH - 1, 0)
        plsc.subcore_barrier()

        def _fin_chunk(j, carry):
            idx = j * NS + s

            @pl.when(idx < NCH)
            def _():
                nb = idx * B
                pltpu.sync_copy(num_sp.at[pl.ds(nb, B)], rb0)
                for h in range(HQ):
                    pltpu.sync_copy(dens[h].at[pl.ds(nb, B)],
                                    pbs[h].at[pl.ds(0, B)])

                def _grp(g, carry2):
                    rv = [1.0 / (pbs[h][pl.ds(g * 16, 16)] + 1e-9)
                          for h in range(HQ)]
                    base = g * 16
                    for ee in range(16):
                        for h in range(HQ):
                            dv = jnp.full((16,), rv[h][ee])
                            for t in range(2):
                                sl = pl.ds(h * 32 + t * 16, 16)
                                qv = rb0[base + ee, sl] * dv + bv[sl]
                                rb0[base + ee, sl] = jnp.maximum(qv, 0.0)
                    return carry2
                lax.fori_loop(0, B // 16, _grp, 0)
                pltpu.sync_copy(rb0, out_hbm.at[q, pl.ds(nb, B)])
            return carry
        lax.fori_loop(0, (NCH + NS - 1) // NS, _fin_chunk, 0)
        plsc.subcore_barrier()


def _conv_sc(feat4, elt, src, dst, cvec, b):
    mesh = plsc.VectorSubcoreMesh(core_axis_name="c", subcore_axis_name="s")
    return pl.kernel(
        _conv_sc_body,
        out_type=jax.ShapeDtypeStruct((4, N, DQ), jnp.float32),
        mesh=mesh,
        compiler_params=pltpu.CompilerParams(needs_layout_passes=False, use_tc_tiling_on_sc=False),
        scratch_types=[
            pltpu.VMEM((2 * HQ * PN,), jnp.float32),
            pltpu.VMEM((EPT,), jnp.int32),
            pltpu.VMEM((EPT,), jnp.int32),
            pltpu.VMEM((B, DQ), jnp.float32),
            pltpu.VMEM((B, DQ), jnp.float32),
            pltpu.VMEM((PB,), jnp.float32),
            pltpu.VMEM((PB,), jnp.float32),
            pltpu.VMEM((B,), jnp.int32),
            pltpu.VMEM((B,), jnp.int32),
            pltpu.VMEM((B,), jnp.int32),
            pltpu.VMEM((B,), jnp.int32),
            pltpu.VMEM((128,), jnp.float32),
            pltpu.VMEM((DQ,), jnp.float32),
            pltpu.VMEM_SHARED((N, DQ), jnp.float32),
            pltpu.VMEM_SHARED((PN,), jnp.float32),
            pltpu.VMEM_SHARED((PN,), jnp.float32),
            pltpu.SemaphoreType.DMA,
            pltpu.SemaphoreType.DMA,
        ],
    )(feat4, elt, src, dst, cvec, b)


def _alr_mat(al, ar):
    d = jnp.arange(D)
    hh = d // F
    ff = d % F
    ml = jnp.zeros((D, H), jnp.float32).at[d, hh].set(al[hh, ff])
    mr = jnp.zeros((D, H), jnp.float32).at[d, hh].set(ar[hh, ff])
    return jnp.concatenate([ml, mr], axis=1)


def _gat_layer(h, src, dst, W, al, ar, b):
    feat, elr, bmax = _proj(h, W, _alr_mat(al, ar))
    m = jnp.max(bmax, axis=(0, 1))
    c8 = jnp.maximum(m[:H] + m[H:], 0.0)
    cvec = jnp.concatenate([c8, jnp.zeros((128 - H,), jnp.float32)])
    return _conv_sc(feat.reshape(4 * N, DQ), elr.T.reshape(2 * H * N),
                    src, dst, cvec, b)


def kernel(x, edge_index, conv_W0, conv_al0, conv_ar0, conv_b0, ln_g0, ln_b0, ff_W1_0, ff_b1_0, ff_W2_0, ff_b2_0, conv_W1, conv_al1, conv_ar1, conv_b1, ln_g1, ln_b1, ff_W1_1, ff_b1_1, ff_W2_1, ff_b2_1):
    src = edge_index[0]
    dst = edge_index[1]
    o = _gat_layer(x, src, dst, conv_W0, conv_al0, conv_ar0, conv_b0)
    h = _ffn(o, ln_g0, ln_b0, ff_W1_0, ff_b1_0, ff_W2_0, ff_b2_0)
    o = _gat_layer(h, src, dst, conv_W1, conv_al1, conv_ar1, conv_b1)
    h = _ffn(o, ln_g1, ln_b1, ff_W1_1, ff_b1_1, ff_W2_1, ff_b2_1)
    return h


# speedup vs baseline: 57.7847x; 1.1087x over previous
import jax
import jax.numpy as jnp
from jax import lax
from jax.experimental import pallas as pl
from jax.experimental.pallas import tpu as pltpu
from jax.experimental.pallas import tpu_sc as plsc

N = 10000
E = 160000
D = 256
H = 8
F = D // H
NC = 2
NS = 16
DH = D // NC
HC = H // NC
EPT = E // NS
B = 80
NCH = N // B
ECH = EPT // B
BLK = 1000
PN = 10112
PB = 128
DQ = D // 4
HQ = 2


def _proj_kernel(x_ref, w_ref, alr_ref, feat_ref, elr_ref, bmax_ref):
    feat = jnp.dot(x_ref[...], w_ref[...], preferred_element_type=jnp.float32)
    feat_ref[...] = feat
    elr = jnp.dot(feat, alr_ref[...], preferred_element_type=jnp.float32)
    elr_ref[...] = elr
    bmax_ref[...] = jnp.max(elr, axis=0, keepdims=True)[None]


def _proj(x, w, alr):
    grid = N // BLK
    return pl.pallas_call(
        _proj_kernel,
        grid=(grid,),
        in_specs=[
            pl.BlockSpec((BLK, D), lambda i: (i, 0)),
            pl.BlockSpec((D, D), lambda i: (0, 0)),
            pl.BlockSpec((D, 2 * H), lambda i: (0, 0)),
        ],
        out_specs=[
            pl.BlockSpec((BLK, D), lambda i: (i, 0)),
            pl.BlockSpec((BLK, 2 * H), lambda i: (i, 0)),
            pl.BlockSpec((1, 1, 2 * H), lambda i: (i, 0, 0)),
        ],
        out_shape=[
            jax.ShapeDtypeStruct((N, D), jnp.float32),
            jax.ShapeDtypeStruct((N, 2 * H), jnp.float32),
            jax.ShapeDtypeStruct((grid, 1, 2 * H), jnp.float32),
        ],
    )(x, w, alr)


def _ffn_kernel(a_ref, b_ref, c_ref, d_ref, g_ref, bn_ref, w1_ref, b1_ref,
                w2_ref, b2_ref, o_ref):
    h = jnp.concatenate([a_ref[...], b_ref[...], c_ref[...], d_ref[...]],
                        axis=1)
    mu = jnp.mean(h, axis=-1, keepdims=True)
    xc = h - mu
    var = jnp.mean(xc * xc, axis=-1, keepdims=True)
    ln = xc * lax.rsqrt(var + 1e-5) * g_ref[...] + bn_ref[...]
    f1 = jnp.maximum(
        jnp.dot(ln, w1_ref[...], preferred_element_type=jnp.float32)
        + b1_ref[...], 0.0)
    f2 = jnp.dot(f1, w2_ref[...], preferred_element_type=jnp.float32) \
        + b2_ref[...]
    o_ref[...] = h + f2


def _ffn(o4, g, bn, w1, b1, w2, b2):
    grid = N // BLK
    vec = pl.BlockSpec((1, D), lambda i: (0, 0))
    mat = pl.BlockSpec((D, D), lambda i: (0, 0))
    quad = pl.BlockSpec((BLK, DQ), lambda i: (i, 0))
    return pl.pallas_call(
        _ffn_kernel,
        grid=(grid,),
        in_specs=[quad, quad, quad, quad, vec, vec, mat, vec, mat, vec],
        out_specs=pl.BlockSpec((BLK, D), lambda i: (i, 0)),
        out_shape=jax.ShapeDtypeStruct((N, D), jnp.float32),
    )(o4[0], o4[1], o4[2], o4[3], g.reshape(1, D), bn.reshape(1, D), w1,
      b1.reshape(1, D), w2, b2.reshape(1, D))


def _conv_sc_body(feat_hbm, elt_hbm, src_hbm, dst_hbm, c_hbm, b_hbm, out_hbm,
                  elv, srcv, dstv, rb0, rb1, pb0, pb1, gi0, gi1, di0, di1,
                  cv, bv, num_sp, dn0, dn1, sem0, sem1, semN0, semN1, semD0, semD1):
    c = lax.axis_index("c")
    s = lax.axis_index("s")
    pbs = [pb0, pb1]
    dens = [dn0, dn1]
    bufs = [(rb0, gi0, di0, sem0, semN0, semD0),
            (rb1, gi1, di1, sem1, semN1, semD1)]
    zero16 = jnp.zeros((16,), jnp.float32)

    pltpu.sync_copy(src_hbm.at[pl.ds(s * EPT, EPT)], srcv)
    pltpu.sync_copy(dst_hbm.at[pl.ds(s * EPT, EPT)], dstv)
    pltpu.sync_copy(c_hbm, cv)

    for k in range(2):
        q = 2 * k + c
        for r in range(HQ):
            pltpu.sync_copy(
                elt_hbm.at[pl.ds((2 * k * HQ) * N + c * HQ * N + r * N, N)],
                elv.at[pl.ds(r * PN, N)])
            pltpu.sync_copy(
                elt_hbm.at[pl.ds((H + 2 * k * HQ) * N + c * HQ * N + r * N, N)],
                elv.at[pl.ds((HQ + r) * PN, N)])
        pltpu.sync_copy(b_hbm.at[pl.ds((2 * k + c) * DQ, DQ)], bv)

        def _zero_row(r, carry):
            for t in range(DQ // 16):
                rb0[r, pl.ds(t * 16, 16)] = zero16
            return carry
        lax.fori_loop(0, B, _zero_row, 0)
        for v in range(B // 16):
            for h in range(HQ):
                pbs[h][pl.ds(v * 16, 16)] = zero16

        def _zero_chunk(j, carry):
            idx = j * NS + s

            @pl.when(idx < NCH)
            def _():
                nb = idx * B
                pltpu.sync_copy(rb0, num_sp.at[pl.ds(nb, B)])
                for h in range(HQ):
                    pltpu.sync_copy(pbs[h].at[pl.ds(0, B)],
                                    dens[h].at[pl.ds(nb, B)])
            return carry
        lax.fori_loop(0, (NCH + NS - 1) // NS, _zero_chunk, 0)
        plsc.subcore_barrier()

        def _build_issue(j, par, wait_num):
            rb, gi, di, sm, smn, smd = bufs[par]
            if wait_num:
                pltpu.make_async_copy(rb, num_sp.at[di], smn).wait()
                for h in range(HQ):
                    pltpu.make_async_copy(pbs[h].at[pl.ds(0, B)],
                                          dens[h].at[di], smd).wait()
            for v in range(B // 16):
                sv = srcv[pl.ds(j * B + v * 16, 16)]
                dv = dstv[pl.ds(j * B + v * 16, 16)]
                gi[pl.ds(v * 16, 16)] = sv * 4 + q
                di[pl.ds(v * 16, 16)] = dv
            pltpu.async_copy(feat_hbm.at[gi], rb, sm)

        def _consume(j, par, first_den):
            rb, gi, di, sm, smn, smd = bufs[par]
            del first_den
            for h in range(HQ):
                ch = plsc.load_gather(
                    cv, [jnp.full((16,), 2 * q + h, jnp.int32)])
                for v in range(B // 16):
                    sv = srcv[pl.ds(j * B + v * 16, 16)]
                    dv = dstv[pl.ds(j * B + v * 16, 16)]
                    el = plsc.load_gather(elv, [sv + h * PN])
                    er = plsc.load_gather(elv, [dv + (HQ + h) * PN])
                    e = el + er
                    e = jnp.where(e >= 0.0, e, 0.2 * e)
                    p = jnp.exp(e - ch)
                    pbs[h][pl.ds(v * 16, 16)] = p
            for h in range(HQ):
                pltpu.async_copy(pbs[h].at[pl.ds(0, B)], dens[h].at[di],
                                 smd, add=True)
            pltpu.make_async_copy(feat_hbm.at[gi], rb, sm).wait()

            def _scale_grp(g, carry2):
                p16 = [pbs[h][pl.ds(g * 16, 16)] for h in range(HQ)]
                base = g * 16
                for ee in range(16):
                    for h in range(HQ):
                        pv = jnp.full((16,), p16[h][ee])
                        for t in range(2):
                            sl = pl.ds(h * 32 + t * 16, 16)
                            rb[base + ee, sl] = rb[base + ee, sl] * pv
                return carry2
            lax.fori_loop(0, B // 16, _scale_grp, 0)
            pltpu.async_copy(rb, num_sp.at[di], smn, add=True)

        _build_issue(0, 0, False)
        _build_issue(1, 1, False)
        _consume(0, 0, True)
        _build_issue(2, 0, True)
        _consume(1, 1, False)

        def _pair(jj, carry):
            j0 = 2 * jj
            _build_issue(j0 + 1, 1, True)
            _consume(j0, 0, False)
            _build_issue(j0 + 2, 0, True)
            _consume(j0 + 1, 1, False)
            return carry
        lax.fori_loop(1, (ECH - 1) // 2, _pair, 0)
        _consume(ECH - 1, 0, False)
        pltpu.make_async_copy(rb0, num_sp.at[di0], semN0).wait()
        pltpu.make_async_copy(rb1, num_sp.at[di1], semN1).wait()
        for h in range(HQ):
            pltpu.make_async_copy(pbs[h].at[pl.ds(0, B)], dens[h].at[di0],
                                  semD0).wait()
            pltpu.make_async_copy(pbs[h].at[pl.ds(0, B)], dens[h].at[di1],
                                  semD1).wait()
        plsc.subcore_barrier()

        def _fin_chunk(j, carry):
            idx = j * NS + s

            @pl.when(idx < NCH)
            def _():
                nb = idx * B
                pltpu.sync_copy(num_sp.at[pl.ds(nb, B)], rb0)
                for h in range(HQ):
                    pltpu.sync_copy(dens[h].at[pl.ds(nb, B)],
                                    pbs[h].at[pl.ds(0, B)])

                def _grp(g, carry2):
                    rv = [1.0 / (pbs[h][pl.ds(g * 16, 16)] + 1e-9)
                          for h in range(HQ)]
                    base = g * 16
                    for ee in range(16):
                        for h in range(HQ):
                            dv = jnp.full((16,), rv[h][ee])
                            for t in range(2):
                                sl = pl.ds(h * 32 + t * 16, 16)
                                qv = rb0[base + ee, sl] * dv + bv[sl]
                                rb0[base + ee, sl] = jnp.maximum(qv, 0.0)
                    return carry2
                lax.fori_loop(0, B // 16, _grp, 0)
                pltpu.sync_copy(rb0, out_hbm.at[q, pl.ds(nb, B)])
            return carry
        lax.fori_loop(0, (NCH + NS - 1) // NS, _fin_chunk, 0)
        plsc.subcore_barrier()


def _conv_sc(feat4, elt, src, dst, cvec, b):
    mesh = plsc.VectorSubcoreMesh(core_axis_name="c", subcore_axis_name="s")
    return pl.kernel(
        _conv_sc_body,
        out_type=jax.ShapeDtypeStruct((4, N, DQ), jnp.float32),
        mesh=mesh,
        compiler_params=pltpu.CompilerParams(needs_layout_passes=False, use_tc_tiling_on_sc=False),
        scratch_types=[
            pltpu.VMEM((2 * HQ * PN,), jnp.float32),
            pltpu.VMEM((EPT,), jnp.int32),
            pltpu.VMEM((EPT,), jnp.int32),
            pltpu.VMEM((B, DQ), jnp.float32),
            pltpu.VMEM((B, DQ), jnp.float32),
            pltpu.VMEM((PB,), jnp.float32),
            pltpu.VMEM((PB,), jnp.float32),
            pltpu.VMEM((B,), jnp.int32),
            pltpu.VMEM((B,), jnp.int32),
            pltpu.VMEM((B,), jnp.int32),
            pltpu.VMEM((B,), jnp.int32),
            pltpu.VMEM((128,), jnp.float32),
            pltpu.VMEM((DQ,), jnp.float32),
            pltpu.VMEM_SHARED((N, DQ), jnp.float32),
            pltpu.VMEM_SHARED((PN,), jnp.float32),
            pltpu.VMEM_SHARED((PN,), jnp.float32),
            pltpu.SemaphoreType.DMA,
            pltpu.SemaphoreType.DMA,
            pltpu.SemaphoreType.DMA,
            pltpu.SemaphoreType.DMA,
            pltpu.SemaphoreType.DMA,
            pltpu.SemaphoreType.DMA,
        ],
    )(feat4, elt, src, dst, cvec, b)


def _alr_mat(al, ar):
    d = jnp.arange(D)
    hh = d // F
    ff = d % F
    ml = jnp.zeros((D, H), jnp.float32).at[d, hh].set(al[hh, ff])
    mr = jnp.zeros((D, H), jnp.float32).at[d, hh].set(ar[hh, ff])
    return jnp.concatenate([ml, mr], axis=1)


def _gat_layer(h, src, dst, W, al, ar, b):
    feat, elr, bmax = _proj(h, W, _alr_mat(al, ar))
    m = jnp.max(bmax, axis=(0, 1))
    c8 = jnp.maximum(m[:H] + m[H:], 0.0)
    cvec = jnp.concatenate([c8, jnp.zeros((128 - H,), jnp.float32)])
    return _conv_sc(feat.reshape(4 * N, DQ), elr.T.reshape(2 * H * N),
                    src, dst, cvec, b)


def kernel(x, edge_index, conv_W0, conv_al0, conv_ar0, conv_b0, ln_g0, ln_b0, ff_W1_0, ff_b1_0, ff_W2_0, ff_b2_0, conv_W1, conv_al1, conv_ar1, conv_b1, ln_g1, ln_b1, ff_W1_1, ff_b1_1, ff_W2_1, ff_b2_1):
    src = edge_index[0]
    dst = edge_index[1]
    o = _gat_layer(x, src, dst, conv_W0, conv_al0, conv_ar0, conv_b0)
    h = _ffn(o, ln_g0, ln_b0, ff_W1_0, ff_b1_0, ff_W2_0, ff_b2_0)
    o = _gat_layer(h, src, dst, conv_W1, conv_al1, conv_ar1, conv_b1)
    h = _ffn(o, ln_g1, ln_b1, ff_W1_1, ff_b1_1, ff_W2_1, ff_b2_1)
    return h


# speedup vs baseline: 60.0040x; 1.0384x over previous
import jax
import jax.numpy as jnp
from jax import lax
from jax.experimental import pallas as pl
from jax.experimental.pallas import tpu as pltpu
from jax.experimental.pallas import tpu_sc as plsc

N = 10000
E = 160000
D = 256
H = 8
F = D // H
NC = 2
NS = 16
DH = D // NC
HC = H // NC
EPT = E // NS
B = 80
NCH = N // B
ECH = EPT // B
BLK = 1000
PN = 10112
PB = 128
DQ = D // 4
HQ = 2


def _proj_kernel(x_ref, w_ref, alr_ref, feat_ref, elt_ref, bmax_ref):
    i = pl.program_id(0)
    feat = jnp.dot(x_ref[...], w_ref[...], preferred_element_type=jnp.float32)
    feat_ref[...] = feat
    elr = jnp.dot(feat, alr_ref[...], preferred_element_type=jnp.float32)
    elt_ref[...] = elr
    bmax = jnp.max(elr, axis=0, keepdims=True)

    @pl.when(i == 0)
    def _():
        bmax_ref[...] = bmax

    @pl.when(i > 0)
    def _():
        bmax_ref[...] = jnp.maximum(bmax_ref[...], bmax)


def _proj(x, w, alr):
    grid = N // BLK
    return pl.pallas_call(
        _proj_kernel,
        grid=(grid,),
        in_specs=[
            pl.BlockSpec((BLK, D), lambda i: (i, 0)),
            pl.BlockSpec((D, D), lambda i: (0, 0)),
            pl.BlockSpec((D, 2 * H), lambda i: (0, 0)),
        ],
        out_specs=[
            pl.BlockSpec((BLK, D), lambda i: (i, 0)),
            pl.BlockSpec((BLK, 2 * H), lambda i: (i, 0)),
            pl.BlockSpec((1, 2 * H), lambda i: (0, 0)),
        ],
        out_shape=[
            jax.ShapeDtypeStruct((N, D), jnp.float32),
            jax.ShapeDtypeStruct((N, 2 * H), jnp.float32),
            jax.ShapeDtypeStruct((1, 2 * H), jnp.float32),
        ],
    )(x, w, alr)


def _ffn_kernel(a_ref, b_ref, c_ref, d_ref, g_ref, bn_ref, w1_ref, b1_ref,
                w2_ref, b2_ref, o_ref):
    h = jnp.concatenate([a_ref[...], b_ref[...], c_ref[...], d_ref[...]],
                        axis=1)
    mu = jnp.mean(h, axis=-1, keepdims=True)
    xc = h - mu
    var = jnp.mean(xc * xc, axis=-1, keepdims=True)
    ln = xc * lax.rsqrt(var + 1e-5) * g_ref[...] + bn_ref[...]
    f1 = jnp.maximum(
        jnp.dot(ln, w1_ref[...], preferred_element_type=jnp.float32)
        + b1_ref[...], 0.0)
    f2 = jnp.dot(f1, w2_ref[...], preferred_element_type=jnp.float32) \
        + b2_ref[...]
    o_ref[...] = h + f2


def _ffn(o4, g, bn, w1, b1, w2, b2):
    grid = N // BLK
    vec = pl.BlockSpec((1, D), lambda i: (0, 0))
    mat = pl.BlockSpec((D, D), lambda i: (0, 0))
    quad = pl.BlockSpec((BLK, DQ), lambda i: (i, 0))
    return pl.pallas_call(
        _ffn_kernel,
        grid=(grid,),
        in_specs=[quad, quad, quad, quad, vec, vec, mat, vec, mat, vec],
        out_specs=pl.BlockSpec((BLK, D), lambda i: (i, 0)),
        out_shape=jax.ShapeDtypeStruct((N, D), jnp.float32),
    )(o4[0], o4[1], o4[2], o4[3], g.reshape(1, D), bn.reshape(1, D), w1,
      b1.reshape(1, D), w2, b2.reshape(1, D))


def _conv_sc_body(feat_hbm, elt_hbm, src_hbm, dst_hbm, c_hbm, b_hbm, out_hbm,
                  elv, srcv, dstv, rb0, rb1, pb0, pb1, gi0, gi1, di0, di1,
                  cv, bv, num_sp, dn0, dn1, sem0, sem1, semN0, semN1, semD0, semD1):
    c = lax.axis_index("c")
    s = lax.axis_index("s")
    pbs = [pb0, pb1]
    dens = [dn0, dn1]
    bufs = [(rb0, gi0, di0, sem0, semN0, semD0),
            (rb1, gi1, di1, sem1, semN1, semD1)]
    zero16 = jnp.zeros((16,), jnp.float32)

    pltpu.sync_copy(src_hbm.at[pl.ds(s * EPT, EPT)], srcv)
    pltpu.sync_copy(dst_hbm.at[pl.ds(s * EPT, EPT)], dstv)
    pltpu.sync_copy(c_hbm, cv)

    for k in range(2):
        q = 2 * k + c
        for r in range(HQ):
            pltpu.sync_copy(
                elt_hbm.at[pl.ds((2 * k * HQ) * N + c * HQ * N + r * N, N)],
                elv.at[pl.ds(r * PN, N)])
            pltpu.sync_copy(
                elt_hbm.at[pl.ds((H + 2 * k * HQ) * N + c * HQ * N + r * N, N)],
                elv.at[pl.ds((HQ + r) * PN, N)])
        pltpu.sync_copy(b_hbm.at[pl.ds((2 * k + c) * DQ, DQ)], bv)

        def _zero_row(r, carry):
            for t in range(DQ // 16):
                rb0[r, pl.ds(t * 16, 16)] = zero16
            return carry
        lax.fori_loop(0, B, _zero_row, 0)
        for v in range(B // 16):
            for h in range(HQ):
                pbs[h][pl.ds(v * 16, 16)] = zero16

        def _zero_chunk(j, carry):
            idx = j * NS + s

            @pl.when(idx < NCH)
            def _():
                nb = idx * B
                pltpu.sync_copy(rb0, num_sp.at[pl.ds(nb, B)])
                for h in range(HQ):
                    pltpu.sync_copy(pbs[h].at[pl.ds(0, B)],
                                    dens[h].at[pl.ds(nb, B)])
            return carry
        lax.fori_loop(0, (NCH + NS - 1) // NS, _zero_chunk, 0)
        plsc.subcore_barrier()

        def _build_issue(j, par, wait_num):
            rb, gi, di, sm, smn, smd = bufs[par]
            if wait_num:
                pltpu.make_async_copy(rb, num_sp.at[di], smn).wait()
                for h in range(HQ):
                    pltpu.make_async_copy(pbs[h].at[pl.ds(0, B)],
                                          dens[h].at[di], smd).wait()
            for v in range(B // 16):
                sv = srcv[pl.ds(j * B + v * 16, 16)]
                dv = dstv[pl.ds(j * B + v * 16, 16)]
                gi[pl.ds(v * 16, 16)] = sv * 4 + q
                di[pl.ds(v * 16, 16)] = dv
            pltpu.async_copy(feat_hbm.at[gi], rb, sm)

        def _consume(j, par, first_den):
            rb, gi, di, sm, smn, smd = bufs[par]
            del first_den
            for h in range(HQ):
                ch = plsc.load_gather(
                    cv, [jnp.full((16,), 2 * q + h, jnp.int32)])
                for v in range(B // 16):
                    sv = srcv[pl.ds(j * B + v * 16, 16)]
                    dv = dstv[pl.ds(j * B + v * 16, 16)]
                    el = plsc.load_gather(elv, [sv + h * PN])
                    er = plsc.load_gather(elv, [dv + (HQ + h) * PN])
                    e = el + er
                    e = jnp.where(e >= 0.0, e, 0.2 * e)
                    p = jnp.exp(e - ch)
                    pbs[h][pl.ds(v * 16, 16)] = p
            for h in range(HQ):
                pltpu.async_copy(pbs[h].at[pl.ds(0, B)], dens[h].at[di],
                                 smd, add=True)
            pltpu.make_async_copy(feat_hbm.at[gi], rb, sm).wait()

            def _scale_grp(g, carry2):
                p16 = [pbs[h][pl.ds(g * 16, 16)] for h in range(HQ)]
                base = g * 16
                for ee in range(16):
                    for h in range(HQ):
                        pv = jnp.full((16,), p16[h][ee])
                        for t in range(2):
                            sl = pl.ds(h * 32 + t * 16, 16)
                            rb[base + ee, sl] = rb[base + ee, sl] * pv
                return carry2
            lax.fori_loop(0, B // 16, _scale_grp, 0)
            pltpu.async_copy(rb, num_sp.at[di], smn, add=True)

        _build_issue(0, 0, False)
        _build_issue(1, 1, False)
        _consume(0, 0, True)
        _build_issue(2, 0, True)
        _consume(1, 1, False)

        def _pair(jj, carry):
            j0 = 2 * jj
            _build_issue(j0 + 1, 1, True)
            _consume(j0, 0, False)
            _build_issue(j0 + 2, 0, True)
            _consume(j0 + 1, 1, False)
            return carry
        lax.fori_loop(1, (ECH - 1) // 2, _pair, 0)
        _consume(ECH - 1, 0, False)
        pltpu.make_async_copy(rb0, num_sp.at[di0], semN0).wait()
        pltpu.make_async_copy(rb1, num_sp.at[di1], semN1).wait()
        for h in range(HQ):
            pltpu.make_async_copy(pbs[h].at[pl.ds(0, B)], dens[h].at[di0],
                                  semD0).wait()
            pltpu.make_async_copy(pbs[h].at[pl.ds(0, B)], dens[h].at[di1],
                                  semD1).wait()
        plsc.subcore_barrier()

        def _fin_chunk(j, carry):
            idx = j * NS + s

            @pl.when(idx < NCH)
            def _():
                nb = idx * B
                pltpu.sync_copy(num_sp.at[pl.ds(nb, B)], rb0)
                for h in range(HQ):
                    pltpu.sync_copy(dens[h].at[pl.ds(nb, B)],
                                    pbs[h].at[pl.ds(0, B)])

                def _grp(g, carry2):
                    rv = [1.0 / (pbs[h][pl.ds(g * 16, 16)] + 1e-9)
                          for h in range(HQ)]
                    base = g * 16
                    for ee in range(16):
                        for h in range(HQ):
                            dv = jnp.full((16,), rv[h][ee])
                            for t in range(2):
                                sl = pl.ds(h * 32 + t * 16, 16)
                                qv = rb0[base + ee, sl] * dv + bv[sl]
                                rb0[base + ee, sl] = jnp.maximum(qv, 0.0)
                    return carry2
                lax.fori_loop(0, B // 16, _grp, 0)
                pltpu.sync_copy(rb0, out_hbm.at[q, pl.ds(nb, B)])
            return carry
        lax.fori_loop(0, (NCH + NS - 1) // NS, _fin_chunk, 0)
        plsc.subcore_barrier()


def _conv_sc(feat4, elt, src, dst, cvec, b):
    mesh = plsc.VectorSubcoreMesh(core_axis_name="c", subcore_axis_name="s")
    return pl.kernel(
        _conv_sc_body,
        out_type=jax.ShapeDtypeStruct((4, N, DQ), jnp.float32),
        mesh=mesh,
        compiler_params=pltpu.CompilerParams(needs_layout_passes=False, use_tc_tiling_on_sc=False),
        scratch_types=[
            pltpu.VMEM((2 * HQ * PN,), jnp.float32),
            pltpu.VMEM((EPT,), jnp.int32),
            pltpu.VMEM((EPT,), jnp.int32),
            pltpu.VMEM((B, DQ), jnp.float32),
            pltpu.VMEM((B, DQ), jnp.float32),
            pltpu.VMEM((PB,), jnp.float32),
            pltpu.VMEM((PB,), jnp.float32),
            pltpu.VMEM((B,), jnp.int32),
            pltpu.VMEM((B,), jnp.int32),
            pltpu.VMEM((B,), jnp.int32),
            pltpu.VMEM((B,), jnp.int32),
            pltpu.VMEM((128,), jnp.float32),
            pltpu.VMEM((DQ,), jnp.float32),
            pltpu.VMEM_SHARED((N, DQ), jnp.float32),
            pltpu.VMEM_SHARED((PN,), jnp.float32),
            pltpu.VMEM_SHARED((PN,), jnp.float32),
            pltpu.SemaphoreType.DMA,
            pltpu.SemaphoreType.DMA,
            pltpu.SemaphoreType.DMA,
            pltpu.SemaphoreType.DMA,
            pltpu.SemaphoreType.DMA,
            pltpu.SemaphoreType.DMA,
        ],
    )(feat4, elt, src, dst, cvec, b)


def _alr_mat(al, ar):
    d = jnp.arange(D)
    sel = (d // F)[:, None] == jnp.arange(H)[None, :]
    ml = jnp.where(sel, al.reshape(D)[:, None], 0.0)
    mr = jnp.where(sel, ar.reshape(D)[:, None], 0.0)
    return jnp.concatenate([ml, mr], axis=1)


def _gat_layer(h, src, dst, W, al, ar, b):
    feat, elt, bmax = _proj(h, W, _alr_mat(al, ar))
    m = bmax.reshape(2 * H)
    c8 = jnp.maximum(m[:H] + m[H:], 0.0)
    cvec = jnp.concatenate([c8, jnp.zeros((128 - H,), jnp.float32)])
    return _conv_sc(feat.reshape(4 * N, DQ), elt.T.reshape(2 * H * N),
                    src, dst, cvec, b)


def kernel(x, edge_index, conv_W0, conv_al0, conv_ar0, conv_b0, ln_g0, ln_b0, ff_W1_0, ff_b1_0, ff_W2_0, ff_b2_0, conv_W1, conv_al1, conv_ar1, conv_b1, ln_g1, ln_b1, ff_W1_1, ff_b1_1, ff_W2_1, ff_b2_1):
    src = edge_index[0]
    dst = edge_index[1]
    o = _gat_layer(x, src, dst, conv_W0, conv_al0, conv_ar0, conv_b0)
    h = _ffn(o, ln_g0, ln_b0, ff_W1_0, ff_b1_0, ff_W2_0, ff_b2_0)
    o = _gat_layer(h, src, dst, conv_W1, conv_al1, conv_ar1, conv_b1)
    h = _ffn(o, ln_g1, ln_b1, ff_W1_1, ff_b1_1, ff_W2_1, ff_b2_1)
    return h


# speedup vs baseline: 61.1343x; 1.0188x over previous
import jax
import jax.numpy as jnp
from jax import lax
from jax.experimental import pallas as pl
from jax.experimental.pallas import tpu as pltpu
from jax.experimental.pallas import tpu_sc as plsc

N = 10000
E = 160000
D = 256
H = 8
F = D // H
NC = 2
NS = 16
DH = D // NC
HC = H // NC
EPT = E // NS
B = 80
NCH = N // B
ECH = EPT // B
BLK = 1000
PN = 10112
PB = 128
DQ = D // 4
HQ = 2


def _proj_kernel(x_ref, w_ref, alr_ref, feat_ref, elt_ref, bmax_ref):
    i = pl.program_id(0)
    feat = jnp.dot(x_ref[...], w_ref[...], preferred_element_type=jnp.float32)
    feat_ref[...] = feat
    elr = jnp.dot(feat, alr_ref[...], preferred_element_type=jnp.float32)
    elt_ref[...] = elr
    bmax = jnp.max(elr, axis=0, keepdims=True)

    @pl.when(i == 0)
    def _():
        bmax_ref[...] = bmax

    @pl.when(i > 0)
    def _():
        bmax_ref[...] = jnp.maximum(bmax_ref[...], bmax)


def _proj(x, w, alr):
    grid = N // BLK
    return pl.pallas_call(
        _proj_kernel,
        grid=(grid,),
        in_specs=[
            pl.BlockSpec((BLK, D), lambda i: (i, 0)),
            pl.BlockSpec((D, D), lambda i: (0, 0)),
            pl.BlockSpec((D, 2 * H), lambda i: (0, 0)),
        ],
        out_specs=[
            pl.BlockSpec((BLK, D), lambda i: (i, 0)),
            pl.BlockSpec((BLK, 2 * H), lambda i: (i, 0)),
            pl.BlockSpec((1, 2 * H), lambda i: (0, 0)),
        ],
        out_shape=[
            jax.ShapeDtypeStruct((N, D), jnp.float32),
            jax.ShapeDtypeStruct((N, 2 * H), jnp.float32),
            jax.ShapeDtypeStruct((1, 2 * H), jnp.float32),
        ],
    )(x, w, alr)


def _ffn_kernel(a_ref, b_ref, c_ref, d_ref, g_ref, bn_ref, w1_ref, b1_ref,
                w2_ref, b2_ref, o_ref):
    h = jnp.concatenate([a_ref[...], b_ref[...], c_ref[...], d_ref[...]],
                        axis=1)
    mu = jnp.mean(h, axis=-1, keepdims=True)
    xc = h - mu
    var = jnp.mean(xc * xc, axis=-1, keepdims=True)
    ln = xc * lax.rsqrt(var + 1e-5) * g_ref[...] + bn_ref[...]
    f1 = jnp.maximum(
        jnp.dot(ln, w1_ref[...], preferred_element_type=jnp.float32)
        + b1_ref[...], 0.0)
    f2 = jnp.dot(f1, w2_ref[...], preferred_element_type=jnp.float32) \
        + b2_ref[...]
    o_ref[...] = h + f2


def _ffn(o4, g, bn, w1, b1, w2, b2):
    grid = N // BLK
    vec = pl.BlockSpec((1, D), lambda i: (0, 0))
    mat = pl.BlockSpec((D, D), lambda i: (0, 0))
    quad = pl.BlockSpec((BLK, DQ), lambda i: (i, 0))
    return pl.pallas_call(
        _ffn_kernel,
        grid=(grid,),
        in_specs=[quad, quad, quad, quad, vec, vec, mat, vec, mat, vec],
        out_specs=pl.BlockSpec((BLK, D), lambda i: (i, 0)),
        out_shape=jax.ShapeDtypeStruct((N, D), jnp.float32),
    )(o4[0], o4[1], o4[2], o4[3], g.reshape(1, D), bn.reshape(1, D), w1,
      b1.reshape(1, D), w2, b2.reshape(1, D))


def _conv_sc_body(feat_hbm, elt_hbm, src_hbm, dst_hbm, c_hbm, b_hbm, out_hbm,
                  elv, srcv, dstv, rb0, rb1, pbc, gi0, gi1, di0, di1,
                  cv, bv, num_sp, dn, sem0, sem1, semN0, semN1, semD0, semD1):
    c = lax.axis_index("c")
    s = lax.axis_index("s")
    bufs = [(rb0, gi0, di0, sem0, semN0, semD0),
            (rb1, gi1, di1, sem1, semN1, semD1)]
    zero16 = jnp.zeros((16,), jnp.float32)

    pltpu.sync_copy(src_hbm.at[pl.ds(s * EPT, EPT)], srcv)
    pltpu.sync_copy(dst_hbm.at[pl.ds(s * EPT, EPT)], dstv)
    pltpu.sync_copy(c_hbm, cv)

    for k in range(2):
        q = 2 * k + c
        for r in range(HQ):
            pltpu.sync_copy(
                elt_hbm.at[pl.ds((2 * k * HQ) * N + c * HQ * N + r * N, N)],
                elv.at[pl.ds(r * PN, N)])
            pltpu.sync_copy(
                elt_hbm.at[pl.ds((H + 2 * k * HQ) * N + c * HQ * N + r * N, N)],
                elv.at[pl.ds((HQ + r) * PN, N)])
        pltpu.sync_copy(b_hbm.at[pl.ds((2 * k + c) * DQ, DQ)], bv)

        def _zero_row(r, carry):
            for t in range(DQ // 16):
                rb0[r, pl.ds(t * 16, 16)] = zero16
            return carry
        lax.fori_loop(0, B, _zero_row, 0)
        for v in range(2 * B // 16):
            pbc[pl.ds(v * 16, 16)] = zero16

        def _zero_chunk(j, carry):
            idx = j * NS + s

            @pl.when(idx < NCH)
            def _():
                nb = idx * B
                pltpu.sync_copy(rb0, num_sp.at[pl.ds(nb, B)])
                for h in range(HQ):
                    pltpu.sync_copy(pbc.at[pl.ds(0, B)],
                                    dn.at[pl.ds(h * PN + nb, B)])
            return carry
        lax.fori_loop(0, (NCH + NS - 1) // NS, _zero_chunk, 0)
        plsc.subcore_barrier()

        def _build_issue(j, par, wait_num):
            rb, gi, di, sm, smn, smd = bufs[par]
            if wait_num:
                pltpu.make_async_copy(rb, num_sp.at[di.at[pl.ds(0, B)]],
                                      smn).wait()
                pltpu.make_async_copy(pbc.at[pl.ds(0, 2 * B)], dn.at[di],
                                      smd).wait()
            for v in range(B // 16):
                sv = srcv[pl.ds(j * B + v * 16, 16)]
                dv = dstv[pl.ds(j * B + v * 16, 16)]
                gi[pl.ds(v * 16, 16)] = sv * 4 + q
                di[pl.ds(v * 16, 16)] = dv
                di[pl.ds(B + v * 16, 16)] = dv + PN
            pltpu.async_copy(feat_hbm.at[gi], rb, sm)

        def _consume(j, par, first_den):
            rb, gi, di, sm, smn, smd = bufs[par]
            del first_den
            chs = [plsc.load_gather(
                cv, [jnp.full((16,), 2 * q + h, jnp.int32)])
                for h in range(HQ)]
            for v in range(B // 16):
                sv = srcv[pl.ds(j * B + v * 16, 16)]
                dv = dstv[pl.ds(j * B + v * 16, 16)]
                for h in range(HQ):
                    el = plsc.load_gather(elv, [sv + h * PN])
                    er = plsc.load_gather(elv, [dv + (HQ + h) * PN])
                    e = el + er
                    e = jnp.where(e >= 0.0, e, 0.2 * e)
                    p = jnp.exp(e - chs[h])
                    pbc[pl.ds(h * B + v * 16, 16)] = p
            pltpu.async_copy(pbc.at[pl.ds(0, 2 * B)], dn.at[di], smd,
                             add=True)
            pltpu.make_async_copy(feat_hbm.at[gi], rb, sm).wait()

            def _scale_grp(g, carry2):
                p16 = [pbc[pl.ds(h * B + g * 16, 16)] for h in range(HQ)]
                base = g * 16
                for ee in range(16):
                    for h in range(HQ):
                        pv = jnp.full((16,), p16[h][ee])
                        for t in range(2):
                            sl = pl.ds(h * 32 + t * 16, 16)
                            rb[base + ee, sl] = rb[base + ee, sl] * pv
                return carry2
            lax.fori_loop(0, B // 16, _scale_grp, 0)
            pltpu.async_copy(rb, num_sp.at[di.at[pl.ds(0, B)]], smn, add=True)

        _build_issue(0, 0, False)
        _build_issue(1, 1, False)
        _consume(0, 0, True)
        _build_issue(2, 0, True)
        _consume(1, 1, False)

        def _pair(jj, carry):
            j0 = 2 * jj
            _build_issue(j0 + 1, 1, True)
            _consume(j0, 0, False)
            _build_issue(j0 + 2, 0, True)
            _consume(j0 + 1, 1, False)
            return carry
        lax.fori_loop(1, (ECH - 1) // 2, _pair, 0)
        _consume(ECH - 1, 0, False)
        pltpu.make_async_copy(rb0, num_sp.at[di0.at[pl.ds(0, B)]],
                              semN0).wait()
        pltpu.make_async_copy(rb1, num_sp.at[di1.at[pl.ds(0, B)]],
                              semN1).wait()
        pltpu.make_async_copy(pbc.at[pl.ds(0, 2 * B)], dn.at[di0],
                              semD0).wait()
        pltpu.make_async_copy(pbc.at[pl.ds(0, 2 * B)], dn.at[di1],
                              semD1).wait()
        plsc.subcore_barrier()

        def _fin_chunk(j, carry):
            idx = j * NS + s

            @pl.when(idx < NCH)
            def _():
                nb = idx * B
                pltpu.sync_copy(num_sp.at[pl.ds(nb, B)], rb0)
                for h in range(HQ):
                    pltpu.sync_copy(dn.at[pl.ds(h * PN + nb, B)],
                                    pbc.at[pl.ds(h * B, B)])

                def _grp(g, carry2):
                    rv = [1.0 / (pbc[pl.ds(h * B + g * 16, 16)] + 1e-9)
                          for h in range(HQ)]
                    base = g * 16
                    for ee in range(16):
                        for h in range(HQ):
                            dv = jnp.full((16,), rv[h][ee])
                            for t in range(2):
                                sl = pl.ds(h * 32 + t * 16, 16)
                                qv = rb0[base + ee, sl] * dv + bv[sl]
                                rb0[base + ee, sl] = jnp.maximum(qv, 0.0)
                    return carry2
                lax.fori_loop(0, B // 16, _grp, 0)
                pltpu.sync_copy(rb0, out_hbm.at[q, pl.ds(nb, B)])
            return carry
        lax.fori_loop(0, (NCH + NS - 1) // NS, _fin_chunk, 0)
        plsc.subcore_barrier()


def _conv_sc(feat4, elt, src, dst, cvec, b):
    mesh = plsc.VectorSubcoreMesh(core_axis_name="c", subcore_axis_name="s")
    return pl.kernel(
        _conv_sc_body,
        out_type=jax.ShapeDtypeStruct((4, N, DQ), jnp.float32),
        mesh=mesh,
        compiler_params=pltpu.CompilerParams(needs_layout_passes=False, use_tc_tiling_on_sc=False),
        scratch_types=[
            pltpu.VMEM((2 * HQ * PN,), jnp.float32),
            pltpu.VMEM((EPT,), jnp.int32),
            pltpu.VMEM((EPT,), jnp.int32),
            pltpu.VMEM((B, DQ), jnp.float32),
            pltpu.VMEM((B, DQ), jnp.float32),
            pltpu.VMEM((2 * PB,), jnp.float32),
            pltpu.VMEM((B,), jnp.int32),
            pltpu.VMEM((B,), jnp.int32),
            pltpu.VMEM((2 * B,), jnp.int32),
            pltpu.VMEM((2 * B,), jnp.int32),
            pltpu.VMEM((128,), jnp.float32),
            pltpu.VMEM((DQ,), jnp.float32),
            pltpu.VMEM_SHARED((N, DQ), jnp.float32),
            pltpu.VMEM_SHARED((2 * PN,), jnp.float32),
            pltpu.SemaphoreType.DMA,
            pltpu.SemaphoreType.DMA,
            pltpu.SemaphoreType.DMA,
            pltpu.SemaphoreType.DMA,
            pltpu.SemaphoreType.DMA,
            pltpu.SemaphoreType.DMA,
        ],
    )(feat4, elt, src, dst, cvec, b)


def _alr_mat(al, ar):
    d = jnp.arange(D)
    sel = (d // F)[:, None] == jnp.arange(H)[None, :]
    ml = jnp.where(sel, al.reshape(D)[:, None], 0.0)
    mr = jnp.where(sel, ar.reshape(D)[:, None], 0.0)
    return jnp.concatenate([ml, mr], axis=1)


def _gat_layer(h, src, dst, W, al, ar, b):
    feat, elt, bmax = _proj(h, W, _alr_mat(al, ar))
    m = bmax.reshape(2 * H)
    c8 = jnp.maximum(m[:H] + m[H:], 0.0)
    cvec = jnp.concatenate([c8, jnp.zeros((128 - H,), jnp.float32)])
    return _conv_sc(feat.reshape(4 * N, DQ), elt.T.reshape(2 * H * N),
                    src, dst, cvec, b)


def kernel(x, edge_index, conv_W0, conv_al0, conv_ar0, conv_b0, ln_g0, ln_b0, ff_W1_0, ff_b1_0, ff_W2_0, ff_b2_0, conv_W1, conv_al1, conv_ar1, conv_b1, ln_g1, ln_b1, ff_W1_1, ff_b1_1, ff_W2_1, ff_b2_1):
    src = edge_index[0]
    dst = edge_index[1]
    o = _gat_layer(x, src, dst, conv_W0, conv_al0, conv_ar0, conv_b0)
    h = _ffn(o, ln_g0, ln_b0, ff_W1_0, ff_b1_0, ff_W2_0, ff_b2_0)
    o = _gat_layer(h, src, dst, conv_W1, conv_al1, conv_ar1, conv_b1)
    h = _ffn(o, ln_g1, ln_b1, ff_W1_1, ff_b1_1, ff_W2_1, ff_b2_1)
    return h


# speedup vs baseline: 62.7337x; 1.0262x over previous
import jax
import jax.numpy as jnp
from jax import lax
from jax.experimental import pallas as pl
from jax.experimental.pallas import tpu as pltpu
from jax.experimental.pallas import tpu_sc as plsc

N = 10000
E = 160000
D = 256
H = 8
F = D // H
NC = 2
NS = 16
DH = D // NC
HC = H // NC
EPT = E // NS
B = 80
EB = 160
ECF = 62
NCH = N // B
ECH = EPT // B
BLK = 1000
PN = 10112
PB = 128
DQ = D // 4
HQ = 2


def _proj_kernel(x_ref, w_ref, alr_ref, feat_ref, elt_ref, bmax_ref):
    i = pl.program_id(0)
    feat = jnp.dot(x_ref[...], w_ref[...], preferred_element_type=jnp.float32)
    feat_ref[...] = feat
    elr = jnp.dot(feat, alr_ref[...], preferred_element_type=jnp.float32)
    elt_ref[...] = elr
    bmax = jnp.max(elr, axis=0, keepdims=True)

    @pl.when(i == 0)
    def _():
        bmax_ref[...] = bmax

    @pl.when(i > 0)
    def _():
        bmax_ref[...] = jnp.maximum(bmax_ref[...], bmax)


def _proj(x, w, alr):
    grid = N // BLK
    return pl.pallas_call(
        _proj_kernel,
        grid=(grid,),
        in_specs=[
            pl.BlockSpec((BLK, D), lambda i: (i, 0)),
            pl.BlockSpec((D, D), lambda i: (0, 0)),
            pl.BlockSpec((D, 2 * H), lambda i: (0, 0)),
        ],
        out_specs=[
            pl.BlockSpec((BLK, D), lambda i: (i, 0)),
            pl.BlockSpec((BLK, 2 * H), lambda i: (i, 0)),
            pl.BlockSpec((1, 2 * H), lambda i: (0, 0)),
        ],
        out_shape=[
            jax.ShapeDtypeStruct((N, D), jnp.float32),
            jax.ShapeDtypeStruct((N, 2 * H), jnp.float32),
            jax.ShapeDtypeStruct((1, 2 * H), jnp.float32),
        ],
    )(x, w, alr)


def _ffn_kernel(a_ref, b_ref, c_ref, d_ref, g_ref, bn_ref, w1_ref, b1_ref,
                w2_ref, b2_ref, o_ref):
    h = jnp.concatenate([a_ref[...], b_ref[...], c_ref[...], d_ref[...]],
                        axis=1)
    mu = jnp.mean(h, axis=-1, keepdims=True)
    xc = h - mu
    var = jnp.mean(xc * xc, axis=-1, keepdims=True)
    ln = xc * lax.rsqrt(var + 1e-5) * g_ref[...] + bn_ref[...]
    f1 = jnp.maximum(
        jnp.dot(ln, w1_ref[...], preferred_element_type=jnp.float32)
        + b1_ref[...], 0.0)
    f2 = jnp.dot(f1, w2_ref[...], preferred_element_type=jnp.float32) \
        + b2_ref[...]
    o_ref[...] = h + f2


def _ffn(o4, g, bn, w1, b1, w2, b2):
    grid = N // BLK
    vec = pl.BlockSpec((1, D), lambda i: (0, 0))
    mat = pl.BlockSpec((D, D), lambda i: (0, 0))
    quad = pl.BlockSpec((BLK, DQ), lambda i: (i, 0))
    return pl.pallas_call(
        _ffn_kernel,
        grid=(grid,),
        in_specs=[quad, quad, quad, quad, vec, vec, mat, vec, mat, vec],
        out_specs=pl.BlockSpec((BLK, D), lambda i: (i, 0)),
        out_shape=jax.ShapeDtypeStruct((N, D), jnp.float32),
    )(o4[0], o4[1], o4[2], o4[3], g.reshape(1, D), bn.reshape(1, D), w1,
      b1.reshape(1, D), w2, b2.reshape(1, D))


def _conv_sc_body(feat_hbm, elt_hbm, src_hbm, dst_hbm, c_hbm, b_hbm, out_hbm,
                  elv, srcv, dstv, rb0, rb1, pbc, gi0, gi1, di0, di1,
                  cv, bv, num_sp, dn, sem0, sem1, semN0, semN1, semD0, semD1):
    c = lax.axis_index("c")
    s = lax.axis_index("s")
    bufs = [(rb0, gi0, di0, sem0, semN0, semD0),
            (rb1, gi1, di1, sem1, semN1, semD1)]
    zero16 = jnp.zeros((16,), jnp.float32)

    pltpu.sync_copy(src_hbm.at[pl.ds(s * EPT, EPT)], srcv)
    pltpu.sync_copy(dst_hbm.at[pl.ds(s * EPT, EPT)], dstv)
    pltpu.sync_copy(c_hbm, cv)

    for k in range(2):
        q = 2 * k + c
        for r in range(HQ):
            pltpu.sync_copy(
                elt_hbm.at[pl.ds((2 * k * HQ) * N + c * HQ * N + r * N, N)],
                elv.at[pl.ds(r * PN, N)])
            pltpu.sync_copy(
                elt_hbm.at[pl.ds((H + 2 * k * HQ) * N + c * HQ * N + r * N, N)],
                elv.at[pl.ds((HQ + r) * PN, N)])
        pltpu.sync_copy(b_hbm.at[pl.ds((2 * k + c) * DQ, DQ)], bv)

        def _zero_row(r, carry):
            for t in range(DQ // 16):
                rb0[r, pl.ds(t * 16, 16)] = zero16
            return carry
        lax.fori_loop(0, B, _zero_row, 0)
        for v in range(2 * EB // 16):
            pbc[pl.ds(v * 16, 16)] = zero16

        def _zero_chunk(j, carry):
            idx = j * NS + s

            @pl.when(idx < NCH)
            def _():
                nb = idx * B
                pltpu.sync_copy(rb0.at[pl.ds(0, B)],
                                num_sp.at[pl.ds(nb, B)])
                for h in range(HQ):
                    pltpu.sync_copy(pbc.at[pl.ds(0, B)],
                                    dn.at[pl.ds(h * PN + nb, B)])
            return carry
        lax.fori_loop(0, (NCH + NS - 1) // NS, _zero_chunk, 0)
        plsc.subcore_barrier()

        def _wait_prev(par):
            rb, gi, di, sm, smn, smd = bufs[par]
            pltpu.make_async_copy(rb, num_sp.at[di.at[pl.ds(0, EB)]],
                                  smn).wait()
            pltpu.make_async_copy(pbc.at[pl.ds(0, 2 * EB)], dn.at[di],
                                  smd).wait()

        def _build_issue(j, par):
            rb, gi, di, sm, smn, smd = bufs[par]
            for v in range(EB // 16):
                sv = srcv[pl.ds(j * EB + v * 16, 16)]
                dv = dstv[pl.ds(j * EB + v * 16, 16)]
                gi[pl.ds(v * 16, 16)] = sv * 4 + q
                di[pl.ds(v * 16, 16)] = dv
                di[pl.ds(EB + v * 16, 16)] = dv + PN
            pltpu.async_copy(feat_hbm.at[gi], rb, sm)

        def _build_issue_tail(par):
            rb, gi, di, sm, smn, smd = bufs[par]
            for v in range(B // 16):
                sv = srcv[pl.ds(ECF * EB + v * 16, 16)]
                dv = dstv[pl.ds(ECF * EB + v * 16, 16)]
                gi[pl.ds(v * 16, 16)] = sv * 4 + q
                di[pl.ds(v * 16, 16)] = dv
                di[pl.ds(B + v * 16, 16)] = dv + PN
            pltpu.async_copy(feat_hbm.at[gi.at[pl.ds(0, B)]],
                             rb.at[pl.ds(0, B)], sm)

        def _consume(j, par):
            rb, gi, di, sm, smn, smd = bufs[par]
            chs = [plsc.load_gather(
                cv, [jnp.full((16,), 2 * q + h, jnp.int32)])
                for h in range(HQ)]
            for v in range(EB // 16):
                sv = srcv[pl.ds(j * EB + v * 16, 16)]
                dv = dstv[pl.ds(j * EB + v * 16, 16)]
                for h in range(HQ):
                    el = plsc.load_gather(elv, [sv + h * PN])
                    er = plsc.load_gather(elv, [dv + (HQ + h) * PN])
                    e = el + er
                    e = jnp.where(e >= 0.0, e, 0.2 * e)
                    p = jnp.exp(e - chs[h])
                    pbc[pl.ds(h * EB + v * 16, 16)] = p
            pltpu.async_copy(pbc.at[pl.ds(0, 2 * EB)], dn.at[di], smd,
                             add=True)
            pltpu.make_async_copy(feat_hbm.at[gi], rb, sm).wait()

            def _scale_grp(g, carry2):
                p16 = [pbc[pl.ds(h * EB + g * 16, 16)] for h in range(HQ)]
                base = g * 16
                for ee in range(16):
                    for h in range(HQ):
                        pv = jnp.full((16,), p16[h][ee])
                        for t in range(2):
                            sl = pl.ds(h * 32 + t * 16, 16)
                            rb[base + ee, sl] = rb[base + ee, sl] * pv
                return carry2
            lax.fori_loop(0, EB // 16, _scale_grp, 0)
            pltpu.async_copy(rb, num_sp.at[di.at[pl.ds(0, EB)]], smn,
                             add=True)

        def _consume_tail(par):
            rb, gi, di, sm, smn, smd = bufs[par]
            chs = [plsc.load_gather(
                cv, [jnp.full((16,), 2 * q + h, jnp.int32)])
                for h in range(HQ)]
            for v in range(B // 16):
                sv = srcv[pl.ds(ECF * EB + v * 16, 16)]
                dv = dstv[pl.ds(ECF * EB + v * 16, 16)]
                for h in range(HQ):
                    el = plsc.load_gather(elv, [sv + h * PN])
                    er = plsc.load_gather(elv, [dv + (HQ + h) * PN])
                    e = el + er
                    e = jnp.where(e >= 0.0, e, 0.2 * e)
                    p = jnp.exp(e - chs[h])
                    pbc[pl.ds(h * B + v * 16, 16)] = p
            pltpu.async_copy(pbc.at[pl.ds(0, 2 * B)],
                             dn.at[di.at[pl.ds(0, 2 * B)]], smd, add=True)
            pltpu.make_async_copy(feat_hbm.at[gi.at[pl.ds(0, B)]],
                                  rb.at[pl.ds(0, B)], sm).wait()

            def _scale_grp(g, carry2):
                p16 = [pbc[pl.ds(h * B + g * 16, 16)] for h in range(HQ)]
                base = g * 16
                for ee in range(16):
                    for h in range(HQ):
                        pv = jnp.full((16,), p16[h][ee])
                        for t in range(2):
                            sl = pl.ds(h * 32 + t * 16, 16)
                            rb[base + ee, sl] = rb[base + ee, sl] * pv
                return carry2
            lax.fori_loop(0, B // 16, _scale_grp, 0)
            pltpu.async_copy(rb.at[pl.ds(0, B)],
                             num_sp.at[di.at[pl.ds(0, B)]], smn, add=True)

        _build_issue(0, 0)
        _build_issue(1, 1)
        _consume(0, 0)
        _wait_prev(0)
        _build_issue(2, 0)
        _consume(1, 1)

        def _pair(jj, carry):
            j0 = 2 * jj
            _wait_prev(1)
            _build_issue(j0 + 1, 1)
            _consume(j0, 0)
            _wait_prev(0)

            @pl.when(jj < ECF // 2 - 1)
            def _():
                _build_issue(j0 + 2, 0)

            @pl.when(jj == ECF // 2 - 1)
            def _():
                _build_issue_tail(0)
            _consume(j0 + 1, 1)
            return carry
        lax.fori_loop(1, ECF // 2, _pair, 0)
        _consume_tail(0)
        pltpu.make_async_copy(rb1, num_sp.at[di1.at[pl.ds(0, EB)]],
                              semN1).wait()
        pltpu.make_async_copy(pbc.at[pl.ds(0, 2 * EB)], dn.at[di1],
                              semD1).wait()
        pltpu.make_async_copy(rb0.at[pl.ds(0, B)],
                              num_sp.at[di0.at[pl.ds(0, B)]], semN0).wait()
        pltpu.make_async_copy(pbc.at[pl.ds(0, 2 * B)],
                              dn.at[di0.at[pl.ds(0, 2 * B)]], semD0).wait()
        plsc.subcore_barrier()

        def _fin_chunk(j, carry):
            idx = j * NS + s

            @pl.when(idx < NCH)
            def _():
                nb = idx * B
                pltpu.sync_copy(num_sp.at[pl.ds(nb, B)],
                                rb0.at[pl.ds(0, B)])
                for h in range(HQ):
                    pltpu.sync_copy(dn.at[pl.ds(h * PN + nb, B)],
                                    pbc.at[pl.ds(h * B, B)])

                def _grp(g, carry2):
                    rv = [1.0 / (pbc[pl.ds(h * B + g * 16, 16)] + 1e-9)
                          for h in range(HQ)]
                    base = g * 16
                    for ee in range(16):
                        for h in range(HQ):
                            dv = jnp.full((16,), rv[h][ee])
                            for t in range(2):
                                sl = pl.ds(h * 32 + t * 16, 16)
                                qv = rb0[base + ee, sl] * dv + bv[sl]
                                rb0[base + ee, sl] = jnp.maximum(qv, 0.0)
                    return carry2
                lax.fori_loop(0, B // 16, _grp, 0)
                pltpu.sync_copy(rb0.at[pl.ds(0, B)],
                                out_hbm.at[q, pl.ds(nb, B)])
            return carry
        lax.fori_loop(0, (NCH + NS - 1) // NS, _fin_chunk, 0)
        plsc.subcore_barrier()


def _conv_sc(feat4, elt, src, dst, cvec, b):
    mesh = plsc.VectorSubcoreMesh(core_axis_name="c", subcore_axis_name="s")
    return pl.kernel(
        _conv_sc_body,
        out_type=jax.ShapeDtypeStruct((4, N, DQ), jnp.float32),
        mesh=mesh,
        compiler_params=pltpu.CompilerParams(needs_layout_passes=False, use_tc_tiling_on_sc=False),
        scratch_types=[
            pltpu.VMEM((2 * HQ * PN,), jnp.float32),
            pltpu.VMEM((EPT,), jnp.int32),
            pltpu.VMEM((EPT,), jnp.int32),
            pltpu.VMEM((EB, DQ), jnp.float32),
            pltpu.VMEM((EB, DQ), jnp.float32),
            pltpu.VMEM((2 * EB,), jnp.float32),
            pltpu.VMEM((EB,), jnp.int32),
            pltpu.VMEM((EB,), jnp.int32),
            pltpu.VMEM((2 * EB,), jnp.int32),
            pltpu.VMEM((2 * EB,), jnp.int32),
            pltpu.VMEM((128,), jnp.float32),
            pltpu.VMEM((DQ,), jnp.float32),
            pltpu.VMEM_SHARED((N, DQ), jnp.float32),
            pltpu.VMEM_SHARED((2 * PN,), jnp.float32),
            pltpu.SemaphoreType.DMA,
            pltpu.SemaphoreType.DMA,
            pltpu.SemaphoreType.DMA,
            pltpu.SemaphoreType.DMA,
            pltpu.SemaphoreType.DMA,
            pltpu.SemaphoreType.DMA,
        ],
    )(feat4, elt, src, dst, cvec, b)


def _alr_mat(al, ar):
    d = jnp.arange(D)
    sel = (d // F)[:, None] == jnp.arange(H)[None, :]
    ml = jnp.where(sel, al.reshape(D)[:, None], 0.0)
    mr = jnp.where(sel, ar.reshape(D)[:, None], 0.0)
    return jnp.concatenate([ml, mr], axis=1)


def _gat_layer(h, src, dst, W, al, ar, b):
    feat, elt, bmax = _proj(h, W, _alr_mat(al, ar))
    m = bmax.reshape(2 * H)
    c8 = jnp.maximum(m[:H] + m[H:], 0.0)
    cvec = jnp.concatenate([c8, jnp.zeros((128 - H,), jnp.float32)])
    return _conv_sc(feat.reshape(4 * N, DQ), elt.T.reshape(2 * H * N),
                    src, dst, cvec, b)


def kernel(x, edge_index, conv_W0, conv_al0, conv_ar0, conv_b0, ln_g0, ln_b0, ff_W1_0, ff_b1_0, ff_W2_0, ff_b2_0, conv_W1, conv_al1, conv_ar1, conv_b1, ln_g1, ln_b1, ff_W1_1, ff_b1_1, ff_W2_1, ff_b2_1):
    src = edge_index[0]
    dst = edge_index[1]
    o = _gat_layer(x, src, dst, conv_W0, conv_al0, conv_ar0, conv_b0)
    h = _ffn(o, ln_g0, ln_b0, ff_W1_0, ff_b1_0, ff_W2_0, ff_b2_0)
    o = _gat_layer(h, src, dst, conv_W1, conv_al1, conv_ar1, conv_b1)
    h = _ffn(o, ln_g1, ln_b1, ff_W1_1, ff_b1_1, ff_W2_1, ff_b2_1)
    return h
